# baseline (speedup 1.0000x reference)
import functools

import jax
import jax.numpy as jnp
from jax import lax
from jax.experimental import pallas as pl
from jax.experimental.pallas import tpu as pltpu
from jax.experimental.pallas import tpu_sc as plsc

N_COMPONENTS = 1024
EMBEDDING_DIM = 64
BETA = 0.25

_NC = 2
_NS = 16
_NW = _NC * _NS


def _vq_block(x_ref, cb_ref, sc_ref, idx_ref, loss_ref, c2_ref):
    i = pl.program_id(0)
    x = x_ref[0]
    cb = cb_ref[...]

    @pl.when(i == 0)
    def _prep():
        c2_ref[...] = jnp.sum(cb * cb, axis=0, keepdims=True)

    sim = jnp.dot(x, cb, preferred_element_type=jnp.float32)
    x2 = jnp.sum(x * x, axis=1, keepdims=True)
    d = x2 + c2_ref[...] - 2.0 * sim
    r = 1.0 / d
    inv = r * r
    rows = jnp.sum(inv, axis=1, keepdims=True)
    imax = jnp.max(inv, axis=1, keepdims=True)
    sc_ref[...] = inv * (1.0 / rows)
    idx = jnp.argmin(d, axis=1).astype(jnp.int32)
    idx_ref[...] = idx.reshape(1, 1, idx.shape[0])
    part = jnp.sum(jax.lax.rsqrt(imax)).reshape(1, 1)

    @pl.when(i == 0)
    def _init():
        loss_ref[...] = jnp.zeros((1, 1), jnp.float32)

    loss_ref[...] += part


def _gather_body(t, rows_per_worker):
    chunks = []
    off = 0
    while off < t:
        w = min(128, t - off)
        chunks.append((off, w))
        off += w

    def body(cbt_hbm, idx_hbm, q_hbm, idx_v, rows_v, sem):
        w = lax.axis_index("s") * _NC + lax.axis_index("c")
        for j in range(rows_per_worker):
            row = w * rows_per_worker + j
            pltpu.sync_copy(idx_hbm.at[row], idx_v)
            cps = [
                pltpu.async_copy(
                    cbt_hbm.at[idx_v.at[pl.ds(off, width)]],
                    rows_v.at[pl.ds(off, width)],
                    sem,
                )
                for off, width in chunks
            ]
            for cp in cps:
                cp.wait()
            pltpu.sync_copy(rows_v, q_hbm.at[row])

    return body


@jax.jit
def kernel(x, codebook):
    b, t, _ = x.shape
    n = b * t
    sc, idx3, loss = pl.pallas_call(
        _vq_block,
        grid=(b,),
        in_specs=[
            pl.BlockSpec((1, t, EMBEDDING_DIM), lambda i: (i, 0, 0)),
            pl.BlockSpec((EMBEDDING_DIM, N_COMPONENTS), lambda i: (0, 0)),
        ],
        out_specs=[
            pl.BlockSpec((t, N_COMPONENTS), lambda i: (i, 0)),
            pl.BlockSpec((1, 1, t), lambda i: (i, 0, 0)),
            pl.BlockSpec((1, 1), lambda i: (0, 0)),
        ],
        out_shape=[
            jax.ShapeDtypeStruct((n, N_COMPONENTS), jnp.float32),
            jax.ShapeDtypeStruct((b, 1, t), jnp.int32),
            jax.ShapeDtypeStruct((1, 1), jnp.float32),
        ],
        scratch_shapes=[pltpu.VMEM((1, N_COMPONENTS), jnp.float32)],
    )(x, codebook)

    idx2 = idx3.reshape(b, t)
    cbt = jnp.concatenate(
        [codebook.T, jnp.zeros((N_COMPONENTS, 128 - EMBEDDING_DIM),
                               jnp.float32)], axis=1)
    mesh = plsc.VectorSubcoreMesh(core_axis_name="c", subcore_axis_name="s",
                                  num_cores=_NC, num_subcores=_NS)
    rows_per_worker = b // _NW
    gather = pl.kernel(
        _gather_body(t, rows_per_worker),
        out_type=jax.ShapeDtypeStruct((b, t, 128), jnp.float32),
        mesh=mesh,
        scratch_types=[
            pltpu.VMEM((t,), jnp.int32),
            pltpu.VMEM((t, 128), jnp.float32),
            pltpu.SemaphoreType.DMA,
        ],
    )
    q = gather(cbt, idx2)[:, :, :EMBEDDING_DIM]
    vq_loss = (1.0 + BETA) * loss[0, 0] / (n * EMBEDDING_DIM)
    return q, sc, vq_loss


# speedup vs baseline: 1.0679x; 1.0679x over previous
import functools

import jax
import jax.numpy as jnp
from jax import lax
from jax.experimental import pallas as pl
from jax.experimental.pallas import tpu as pltpu
from jax.experimental.pallas import tpu_sc as plsc

N_COMPONENTS = 1024
EMBEDDING_DIM = 64
BETA = 0.25

_NC = 2
_NS = 16
_NW = _NC * _NS


def _vq_block(x_ref, cb_ref, sc_ref, idx_ref, loss_ref, c2_ref):
    i = pl.program_id(0)
    x = x_ref[0]
    cb = cb_ref[...]

    @pl.when(i == 0)
    def _prep():
        c2_ref[...] = jnp.sum(cb * cb, axis=0, keepdims=True)

    sim = jnp.dot(x, cb, preferred_element_type=jnp.float32)
    x2 = jnp.sum(x * x, axis=1, keepdims=True)
    d = x2 + c2_ref[...] - 2.0 * sim
    r = 1.0 / d
    inv = r * r
    rows = jnp.sum(inv, axis=1, keepdims=True)
    imax = jnp.max(inv, axis=1, keepdims=True)
    sc_ref[...] = inv * (1.0 / rows)
    idx = jnp.argmin(d, axis=1).astype(jnp.int32)
    idx_ref[...] = idx.reshape(1, 1, idx.shape[0])
    part = jnp.sum(jax.lax.rsqrt(imax)).reshape(1, 1)

    @pl.when(i == 0)
    def _init():
        loss_ref[...] = jnp.zeros((1, 1), jnp.float32)

    loss_ref[...] += part


def _gather_body(t, rows_per_worker):
    chunks = []
    off = 0
    while off < t:
        w = min(128, t - off)
        chunks.append((off, w))
        off += w

    def body(cbt_hbm, idx_hbm, q_hbm, idx_v, rows_v, sem):
        w = lax.axis_index("s") * _NC + lax.axis_index("c")
        for j in range(rows_per_worker):
            row = w * rows_per_worker + j
            pltpu.sync_copy(idx_hbm.at[row], idx_v)
            cps = [
                pltpu.async_copy(
                    cbt_hbm.at[idx_v.at[pl.ds(off, width)]],
                    rows_v.at[pl.ds(off, width)],
                    sem,
                )
                for off, width in chunks
            ]
            for cp in cps:
                cp.wait()
            pltpu.sync_copy(rows_v, q_hbm.at[row])

    return body


@jax.jit
def kernel(x, codebook):
    b, t, _ = x.shape
    n = b * t
    sc, idx3, loss = pl.pallas_call(
        _vq_block,
        grid=(b,),
        in_specs=[
            pl.BlockSpec((1, t, EMBEDDING_DIM), lambda i: (i, 0, 0)),
            pl.BlockSpec((EMBEDDING_DIM, N_COMPONENTS), lambda i: (0, 0)),
        ],
        out_specs=[
            pl.BlockSpec((t, N_COMPONENTS), lambda i: (i, 0)),
            pl.BlockSpec((1, 1, t), lambda i: (i, 0, 0)),
            pl.BlockSpec((1, 1), lambda i: (0, 0)),
        ],
        out_shape=[
            jax.ShapeDtypeStruct((n, N_COMPONENTS), jnp.float32),
            jax.ShapeDtypeStruct((b, 1, t), jnp.int32),
            jax.ShapeDtypeStruct((1, 1), jnp.float32),
        ],
        scratch_shapes=[pltpu.VMEM((1, N_COMPONENTS), jnp.float32)],
    )(x, codebook)

    idx2 = idx3.reshape(b, t)
    cbt = codebook.T
    mesh = plsc.VectorSubcoreMesh(core_axis_name="c", subcore_axis_name="s",
                                  num_cores=_NC, num_subcores=_NS)
    rows_per_worker = b // _NW
    gather = pl.kernel(
        _gather_body(t, rows_per_worker),
        out_type=jax.ShapeDtypeStruct((b, t, EMBEDDING_DIM), jnp.float32),
        mesh=mesh,
        scratch_types=[
            pltpu.VMEM((t,), jnp.int32),
            pltpu.VMEM((t, EMBEDDING_DIM), jnp.float32),
            pltpu.SemaphoreType.DMA,
        ],
        compiler_params=pltpu.CompilerParams(use_tc_tiling_on_sc=False),
    )
    q = gather(cbt, idx2)
    vq_loss = (1.0 + BETA) * loss[0, 0] / (n * EMBEDDING_DIM)
    return q, sc, vq_loss


# speedup vs baseline: 1.2189x; 1.1414x over previous
import functools

import jax
import jax.numpy as jnp
from jax.experimental import pallas as pl
from jax.experimental.pallas import tpu as pltpu

N_COMPONENTS = 1024
EMBEDDING_DIM = 64
BETA = 0.25


def _vq_block(x_ref, cb_ref, q_ref, sc_ref, loss_ref):
    x = x_ref[0]
    cb = cb_ref[...]
    c2 = jnp.sum(cb * cb, axis=0, keepdims=True)
    sim = jnp.dot(x, cb, preferred_element_type=jnp.float32)
    x2 = jnp.sum(x * x, axis=1, keepdims=True)
    d = x2 + c2 - 2.0 * sim
    r = 1.0 / d
    inv = r * r
    rows = jnp.sum(inv, axis=1, keepdims=True)
    imax = jnp.max(inv, axis=1, keepdims=True)
    sc_ref[...] = inv * (1.0 / rows)
    idx = jnp.argmin(d, axis=1)
    onehot = (jax.lax.broadcasted_iota(jnp.int32, d.shape, 1)
              == idx[:, None]).astype(jnp.float32)
    q = jax.lax.dot_general(onehot, cb, (((1,), (1,)), ((), ())),
                            preferred_element_type=jnp.float32)
    q_ref[0] = q
    loss_ref[...] = jnp.sum(jax.lax.rsqrt(imax)).reshape(1, 1, 1)


@jax.jit
def kernel(x, codebook):
    b, t, _ = x.shape
    n = b * t
    q, sc, loss = pl.pallas_call(
        _vq_block,
        grid=(b,),
        in_specs=[
            pl.BlockSpec((1, t, EMBEDDING_DIM), lambda i: (i, 0, 0)),
            pl.BlockSpec((EMBEDDING_DIM, N_COMPONENTS), lambda i: (0, 0)),
        ],
        out_specs=[
            pl.BlockSpec((1, t, EMBEDDING_DIM), lambda i: (i, 0, 0)),
            pl.BlockSpec((t, N_COMPONENTS), lambda i: (i, 0)),
            pl.BlockSpec((1, 1, 1), lambda i: (i, 0, 0)),
        ],
        out_shape=[
            jax.ShapeDtypeStruct((b, t, EMBEDDING_DIM), jnp.float32),
            jax.ShapeDtypeStruct((n, N_COMPONENTS), jnp.float32),
            jax.ShapeDtypeStruct((b, 1, 1), jnp.float32),
        ],
        compiler_params=pltpu.CompilerParams(
            dimension_semantics=("parallel",)),
    )(x, codebook)
    vq_loss = (1.0 + BETA) * jnp.sum(loss) / (n * EMBEDDING_DIM)
    return q, sc, vq_loss


# speedup vs baseline: 1.7379x; 1.4258x over previous
import functools

import jax
import jax.numpy as jnp
from jax.experimental import pallas as pl
from jax.experimental.pallas import tpu as pltpu

N_COMPONENTS = 1024
EMBEDDING_DIM = 64
BETA = 0.25


def _vq_block(x_ref, cb_ref, q_ref, sc_ref, loss_ref):
    x = x_ref[0]
    cb = cb_ref[...]
    c2 = jnp.sum(cb * cb, axis=0, keepdims=True)
    sim = jnp.dot(x, cb, preferred_element_type=jnp.float32)
    x2 = jnp.sum(x * x, axis=1, keepdims=True)
    d = x2 + c2 - 2.0 * sim
    r = 1.0 / d
    inv = r * r
    rows = jnp.sum(inv, axis=1, keepdims=True)
    dmin = jnp.min(d, axis=1, keepdims=True)
    sc_ref[...] = inv * (1.0 / rows)
    onehot = (d == dmin).astype(jnp.float32)
    q = jax.lax.dot_general(onehot, cb, (((1,), (1,)), ((), ())),
                            preferred_element_type=jnp.float32)
    q_ref[0] = q
    loss_ref[...] = jnp.sum(dmin).reshape(1, 1, 1)


@jax.jit
def kernel(x, codebook):
    b, t, _ = x.shape
    n = b * t
    q, sc, loss = pl.pallas_call(
        _vq_block,
        grid=(b,),
        in_specs=[
            pl.BlockSpec((1, t, EMBEDDING_DIM), lambda i: (i, 0, 0)),
            pl.BlockSpec((EMBEDDING_DIM, N_COMPONENTS), lambda i: (0, 0)),
        ],
        out_specs=[
            pl.BlockSpec((1, t, EMBEDDING_DIM), lambda i: (i, 0, 0)),
            pl.BlockSpec((t, N_COMPONENTS), lambda i: (i, 0)),
            pl.BlockSpec((1, 1, 1), lambda i: (i, 0, 0)),
        ],
        out_shape=[
            jax.ShapeDtypeStruct((b, t, EMBEDDING_DIM), jnp.float32),
            jax.ShapeDtypeStruct((n, N_COMPONENTS), jnp.float32),
            jax.ShapeDtypeStruct((b, 1, 1), jnp.float32),
        ],
        compiler_params=pltpu.CompilerParams(
            dimension_semantics=("parallel",)),
    )(x, codebook)
    vq_loss = (1.0 + BETA) * jnp.sum(loss) / (n * EMBEDDING_DIM)
    return q, sc, vq_loss
